# baseline (speedup 1.0000x reference)
import functools

import jax
import jax.numpy as jnp
from jax import lax
from jax.experimental import pallas as pl
from jax.experimental.pallas import tpu as pltpu
from jax.experimental.pallas import tpu_sc as plsc

_V = 1000
_D = 16
_B = 3_200_000

_info = plsc.get_sparse_core_info()
_NC = _info.num_cores
_NS = _info.num_subcores
_NW = _NC * _NS
_BPW = _B // _NW
_C = 2000
_NCHUNK = _BPW // _C
_BLK = _C // 16
_UNROLL = 5

_mesh = plsc.VectorSubcoreMesh(core_axis_name="c", subcore_axis_name="s")


@functools.partial(
    pl.kernel,
    mesh=_mesh,
    out_type=jax.ShapeDtypeStruct((_B, _D), jnp.float32),
    scratch_types=[
        pltpu.VMEM((_C,), jnp.int32),
        pltpu.VMEM((_C,), jnp.int32),
        pltpu.VMEM((_C, _D), jnp.float32),
        pltpu.VMEM((_C, _D), jnp.float32),
        pltpu.VMEM((_V, _D), jnp.float32),
        pltpu.SemaphoreType.DMA,
        pltpu.SemaphoreType.DMA,
        pltpu.SemaphoreType.DMA,
        pltpu.SemaphoreType.DMA,
    ],
    compiler_params=pltpu.CompilerParams(
        use_tc_tiling_on_sc=False, needs_layout_passes=False
    ),
)
def _lookup(idx_hbm, table_hbm, out_hbm, idx_a, idx_b, rows_a, rows_b,
            table_v, si0, si1, so0, so1):
    idx = (idx_a, idx_b)
    rows = (rows_a, rows_b)
    si = (si0, si1)
    so = (so0, so1)
    sid = lax.axis_index("s")
    wid = sid * _NC + lax.axis_index("c")
    base = wid * _BPW

    pltpu.sync_copy(table_hbm, table_v)

    def istart(g, b):
        pltpu.async_copy(idx_hbm.at[pl.ds(base + g * _C, _C)], idx[b], si[b])

    def iwait(g, b):
        pltpu.make_async_copy(
            idx_hbm.at[pl.ds(base + g * _C, _C)], idx[b], si[b]
        ).wait()

    def ostart(g, b):
        pltpu.async_copy(rows[b], out_hbm.at[pl.ds(base + g * _C, _C)], so[b])

    def owait(g, b):
        pltpu.make_async_copy(
            rows[b], out_hbm.at[pl.ds(base + g * _C, _C)], so[b]
        ).wait()

    iota = lax.iota(jnp.int32, 16)
    cols = [jnp.full((16,), c, jnp.int32) for c in range(_D)]

    def gather_chunk(b):
        def blocks(j, carry):
            for u in range(_UNROLL):
                r0 = (j * _UNROLL + u) * 16
                idxv = idx[b][pl.ds(r0, 16)]
                rowids = iota + r0
                for c in range(_D):
                    vals = plsc.load_gather(table_v, [idxv, cols[c]])
                    plsc.store_scatter(rows[b], [rowids, cols[c]], vals)
            return carry

        lax.fori_loop(0, _BLK // _UNROLL, blocks, 0)

    istart(0, 0)
    istart(1, 1)

    def pair(gp, carry):
        g0 = gp * 2

        @pl.when(gp > 0)
        def _():
            owait(g0 - 2, 0)
        iwait(g0, 0)
        gather_chunk(0)
        ostart(g0, 0)

        @pl.when(g0 + 2 < _NCHUNK)
        def _():
            istart(g0 + 2, 0)

        @pl.when(gp > 0)
        def _():
            owait(g0 - 1, 1)
        iwait(g0 + 1, 1)
        gather_chunk(1)
        ostart(g0 + 1, 1)

        @pl.when(g0 + 3 < _NCHUNK)
        def _():
            istart(g0 + 3, 1)
        return carry

    lax.fori_loop(0, _NCHUNK // 2, pair, 0)
    owait(_NCHUNK - 2, 0)
    owait(_NCHUNK - 1, 1)


def kernel(type_indices, type_embedding_weight):
    return _lookup(type_indices, type_embedding_weight)


# speedup vs baseline: 1.5238x; 1.5238x over previous
import functools

import jax
import jax.numpy as jnp
from jax import lax
from jax.experimental import pallas as pl
from jax.experimental.pallas import tpu as pltpu
from jax.experimental.pallas import tpu_sc as plsc

_V = 1000
_D = 16
_B = 3_200_000

_info = plsc.get_sparse_core_info()
_NC = _info.num_cores
_NS = _info.num_subcores
_NW = _NC * _NS
_BPW = _B // _NW
_C = 2000
_NCHUNK = _BPW // _C
_UNROLL = 2

_mesh = plsc.VectorSubcoreMesh(core_axis_name="c", subcore_axis_name="s")


@functools.partial(
    pl.kernel,
    mesh=_mesh,
    out_type=jax.ShapeDtypeStruct((_B, _D), jnp.float32),
    scratch_types=[
        pltpu.VMEM((_C,), jnp.int32),
        pltpu.VMEM((_C,), jnp.int32),
        pltpu.VMEM((_C, _D), jnp.float32),
        pltpu.VMEM((_C, _D), jnp.float32),
        pltpu.VMEM((_V, _D), jnp.float32),
        pltpu.SemaphoreType.DMA,
        pltpu.SemaphoreType.DMA,
        pltpu.SemaphoreType.DMA,
        pltpu.SemaphoreType.DMA,
    ],
    compiler_params=pltpu.CompilerParams(
        use_tc_tiling_on_sc=False, needs_layout_passes=False
    ),
)
def _lookup(idx_hbm, table_hbm, out_hbm, idx_a, idx_b, rows_a, rows_b,
            table_v, si0, si1, so0, so1):
    idx = (idx_a, idx_b)
    rows = (rows_a, rows_b)
    si = (si0, si1)
    so = (so0, so1)
    sid = lax.axis_index("s")
    wid = sid * _NC + lax.axis_index("c")
    base = wid * _BPW

    pltpu.sync_copy(table_hbm, table_v)

    def istart(g, b):
        pltpu.async_copy(idx_hbm.at[pl.ds(base + g * _C, _C)], idx[b], si[b])

    def iwait(g, b):
        pltpu.make_async_copy(
            idx_hbm.at[pl.ds(base + g * _C, _C)], idx[b], si[b]
        ).wait()

    def ostart(g, b):
        pltpu.async_copy(rows[b], out_hbm.at[pl.ds(base + g * _C, _C)], so[b])

    def owait(g, b):
        pltpu.make_async_copy(
            rows[b], out_hbm.at[pl.ds(base + g * _C, _C)], so[b]
        ).wait()

    def gather_chunk(b):
        @plsc.parallel_loop(0, _C, step=16, unroll=_UNROLL)
        def _(i):
            idxv = idx[b][pl.ds(i, 16)]
            for u in range(16):
                rows[b][i + u, :] = table_v[idxv[u], :]

    istart(0, 0)
    istart(1, 1)

    def pair(gp, carry):
        g0 = gp * 2

        @pl.when(gp > 0)
        def _():
            owait(g0 - 2, 0)
        iwait(g0, 0)
        gather_chunk(0)
        ostart(g0, 0)

        @pl.when(g0 + 2 < _NCHUNK)
        def _():
            istart(g0 + 2, 0)

        @pl.when(gp > 0)
        def _():
            owait(g0 - 1, 1)
        iwait(g0 + 1, 1)
        gather_chunk(1)
        ostart(g0 + 1, 1)

        @pl.when(g0 + 3 < _NCHUNK)
        def _():
            istart(g0 + 3, 1)
        return carry

    lax.fori_loop(0, _NCHUNK // 2, pair, 0)
    owait(_NCHUNK - 2, 0)
    owait(_NCHUNK - 1, 1)


def kernel(type_indices, type_embedding_weight):
    return _lookup(type_indices, type_embedding_weight)
